# Initial kernel scaffold; baseline (speedup 1.0000x reference)
import functools

import jax
import jax.numpy as jnp
from jax import lax
from jax.experimental import pallas as pl
from jax.experimental.pallas import tpu as pltpu
from jax.experimental.pallas import tpu_sc as plsc

N = 10000
M = 5000
NNZ = 320000
D = 128

NC = 2
NS = 16
NW = NC * NS

NP = 10016
MP = 5008
CHUNK = 128
IDXROWS = 2560
NNZP = IDXROWS * CHUNK
ROWS_PER_W = IDXROWS // NW
DEGW = 16

_MESH = dict(core_axis_name="c", subcore_axis_name="s")


def _make_sc_pass(T, S):
    rpt = T // NS
    mesh = plsc.VectorSubcoreMesh(**_MESH)

    @functools.partial(
        pl.kernel,
        out_type=jax.ShapeDtypeStruct((2 * T, D), jnp.float32),
        mesh=mesh,
        scratch_types=[
            pltpu.VMEM((ROWS_PER_W, CHUNK), jnp.int32),
            pltpu.VMEM((ROWS_PER_W, CHUNK), jnp.int32),
            pltpu.VMEM((CHUNK, D), jnp.float32),
            pltpu.VMEM_SHARED((T, D), jnp.float32),
        ],
    )
    def sc_pass(src, gidx, sidx, zeros, out, gbuf, sbuf, rows, acc):
        cid = lax.axis_index("c")
        sid = lax.axis_index("s")
        wid = sid * NC + cid
        pltpu.sync_copy(zeros.at[pl.ds(sid * rpt, rpt)],
                        acc.at[pl.ds(sid * rpt, rpt)])
        pltpu.sync_copy(gidx.at[pl.ds(wid * ROWS_PER_W, ROWS_PER_W)], gbuf)
        pltpu.sync_copy(sidx.at[pl.ds(wid * ROWS_PER_W, ROWS_PER_W)], sbuf)
        plsc.subcore_barrier()

        def body(j, carry):
            pltpu.sync_copy(src.at[gbuf.at[j]], rows)
            pltpu.sync_copy(rows, acc.at[sbuf.at[j]], add=True)
            return carry

        lax.fori_loop(0, ROWS_PER_W, body, 0)
        plsc.subcore_barrier()
        pltpu.sync_copy(acc.at[pl.ds(sid * rpt, rpt)],
                        out.at[pl.ds(cid * T + sid * rpt, rpt)])

    return sc_pass


def _make_sc_degrees():
    rptn = NP // NS
    rptm = MP // NS
    mesh = plsc.VectorSubcoreMesh(**_MESH)

    @functools.partial(
        pl.kernel,
        out_type=(jax.ShapeDtypeStruct((2 * NP, DEGW), jnp.float32),
                  jax.ShapeDtypeStruct((2 * MP, DEGW), jnp.float32)),
        mesh=mesh,
        scratch_types=[
            pltpu.VMEM((ROWS_PER_W, CHUNK), jnp.int32),
            pltpu.VMEM((ROWS_PER_W, CHUNK), jnp.int32),
            pltpu.VMEM((CHUNK, DEGW), jnp.float32),
            pltpu.VMEM_SHARED((NP, DEGW), jnp.float32),
            pltpu.VMEM_SHARED((MP, DEGW), jnp.float32),
        ],
    )
    def sc_degrees(vidx, eidx, ones, zn, zm, dv_out, de_out,
                   vbuf, ebuf, onesbuf, dvacc, deacc):
        cid = lax.axis_index("c")
        sid = lax.axis_index("s")
        wid = sid * NC + cid
        pltpu.sync_copy(zn.at[pl.ds(sid * rptn, rptn)],
                        dvacc.at[pl.ds(sid * rptn, rptn)])
        pltpu.sync_copy(zm.at[pl.ds(sid * rptm, rptm)],
                        deacc.at[pl.ds(sid * rptm, rptm)])
        pltpu.sync_copy(vidx.at[pl.ds(wid * ROWS_PER_W, ROWS_PER_W)], vbuf)
        pltpu.sync_copy(eidx.at[pl.ds(wid * ROWS_PER_W, ROWS_PER_W)], ebuf)
        pltpu.sync_copy(ones, onesbuf)
        plsc.subcore_barrier()

        def body(j, carry):
            pltpu.sync_copy(onesbuf, dvacc.at[vbuf.at[j]], add=True)
            pltpu.sync_copy(onesbuf, deacc.at[ebuf.at[j]], add=True)
            return carry

        lax.fori_loop(0, ROWS_PER_W, body, 0)
        plsc.subcore_barrier()
        pltpu.sync_copy(dvacc.at[pl.ds(sid * rptn, rptn)],
                        dv_out.at[pl.ds(cid * NP + sid * rptn, rptn)])
        pltpu.sync_copy(deacc.at[pl.ds(sid * rptm, rptm)],
                        de_out.at[pl.ds(cid * MP + sid * rptm, rptm)])

    return sc_degrees


def _tc_prep_body(x, w, b, dv2, hs, dvis):
    dv = dv2[0] + dv2[1]
    s = jnp.where(dv > 0, lax.rsqrt(dv), 0.0)
    h = jnp.dot(x[...], w[...], preferred_element_type=jnp.float32) + b[...]
    hs[...] = h * s
    dvis[...] = s


def _tc_prep(X, W1, b1, dv2):
    return pl.pallas_call(
        _tc_prep_body,
        out_shape=(jax.ShapeDtypeStruct((N, D), jnp.float32),
                   jax.ShapeDtypeStruct((N, 1), jnp.float32)),
    )(X, W1, b1, dv2)


def _tc_edge_body(p2, de2, out):
    de = de2[0] + de2[1]
    dei = jnp.where(de > 0, 1.0 / de, 0.0)
    out[...] = (p2[0] + p2[1]) * dei


def _tc_edge(p2, de2):
    return pl.pallas_call(
        _tc_edge_body,
        out_shape=jax.ShapeDtypeStruct((MP, D), jnp.float32),
    )(p2, de2)


def _tc_l2_body(q2, dvis, w, b, out):
    t = jnp.maximum((q2[0] + q2[1]) * dvis[...], 0.0)
    h2 = jnp.dot(t, w[...], preferred_element_type=jnp.float32) + b[...]
    out[...] = h2 * dvis[...]


def _tc_l2(q2, dvis, W2, b2):
    return pl.pallas_call(
        _tc_l2_body,
        out_shape=jax.ShapeDtypeStruct((N, D), jnp.float32),
    )(q2, dvis, W2, b2)


def _tc_final_body(q2, dvis, wvt, bv, out):
    t = jnp.maximum((q2[0] + q2[1]) * dvis[...], 0.0)
    pooled = jnp.sum(t, axis=0, keepdims=True) * (1.0 / N)
    out[...] = jnp.sum(pooled * wvt[...], axis=1, keepdims=True) + bv[...]


def _tc_final(q2, dvis, wvt, bv):
    return pl.pallas_call(
        _tc_final_body,
        out_shape=jax.ShapeDtypeStruct((1, 1), jnp.float32),
    )(q2, dvis, wvt, bv)


_sc_degrees = _make_sc_degrees()
_sc_pass_edge = _make_sc_pass(MP, NP)
_sc_pass_vert = _make_sc_pass(NP, MP)


def kernel(X, vertex_idx, edge_idx, W1, b1, W2, b2, Wv, bv):
    vi = vertex_idx.astype(jnp.int32)
    ei = edge_idx.astype(jnp.int32)
    vip = jnp.concatenate(
        [vi, jnp.full((NNZP - NNZ,), N, jnp.int32)]).reshape(IDXROWS, CHUNK)
    eip = jnp.concatenate(
        [ei, jnp.full((NNZP - NNZ,), M, jnp.int32)]).reshape(IDXROWS, CHUNK)
    ones_deg = jnp.ones((CHUNK, DEGW), jnp.float32)
    zN = jnp.zeros((NP, D), jnp.float32)
    zM = jnp.zeros((MP, D), jnp.float32)
    zNd = jnp.zeros((NP, DEGW), jnp.float32)
    zMd = jnp.zeros((MP, DEGW), jnp.float32)

    dv_out, de_out = _sc_degrees(vip, eip, ones_deg, zNd, zMd)
    dv2 = dv_out.reshape(2, NP, DEGW)[:, :N, :1]
    de2 = de_out.reshape(2, MP, DEGW)[:, :, :1]

    hs, dvis = _tc_prep(X, W1, b1.reshape(1, D), dv2)
    hsp = jnp.pad(hs, ((0, NP - N), (0, 0)))
    p = _sc_pass_edge(hsp, vip, eip, zM).reshape(2, MP, D)
    ef = _tc_edge(p, de2)
    q = _sc_pass_vert(ef, eip, vip, zN).reshape(2, NP, D)[:, :N]

    h2 = _tc_l2(q, dvis, W2, b2.reshape(1, D))
    h2p = jnp.pad(h2, ((0, NP - N), (0, 0)))
    p2 = _sc_pass_edge(h2p, vip, eip, zM).reshape(2, MP, D)
    ef2 = _tc_edge(p2, de2)
    q2 = _sc_pass_vert(ef2, eip, vip, zN).reshape(2, NP, D)[:, :N]

    v = _tc_final(q2, dvis, Wv[:, 0].reshape(1, D), bv.reshape(1, 1))
    return v[0]


# speedup vs baseline: 2.5558x; 2.5558x over previous
import functools

import jax
import jax.numpy as jnp
from jax import lax
from jax.experimental import pallas as pl
from jax.experimental.pallas import tpu as pltpu
from jax.experimental.pallas import tpu_sc as plsc

N = 10000
M = 5000
NNZ = 320000
D = 128

NC = 2
NS = 16
NW = NC * NS

NP = 10112
MP = 5120
CHUNK = 128
IDXROWS = 2560
NNZP = IDXROWS * CHUNK
ROWS_PER_W = IDXROWS // NW
DEGW = 16

_MESH = dict(core_axis_name="c", subcore_axis_name="s")


def _make_sc_pass(T, S):
    rpt = T // NS
    mesh = plsc.VectorSubcoreMesh(**_MESH)

    @functools.partial(
        pl.kernel,
        out_type=jax.ShapeDtypeStruct((2 * T, D), jnp.float32),
        mesh=mesh,
        scratch_types=[
            pltpu.VMEM((ROWS_PER_W, CHUNK), jnp.int32),
            pltpu.VMEM((ROWS_PER_W, CHUNK), jnp.int32),
            pltpu.VMEM((CHUNK, D), jnp.float32),
            pltpu.VMEM_SHARED((T, D), jnp.float32),
        ],
    )
    def sc_pass(src, gidx, sidx, zeros, out, gbuf, sbuf, rows, acc):
        cid = lax.axis_index("c")
        sid = lax.axis_index("s")
        wid = sid * NC + cid
        pltpu.sync_copy(zeros.at[pl.ds(sid * rpt, rpt)],
                        acc.at[pl.ds(sid * rpt, rpt)])
        pltpu.sync_copy(gidx.at[pl.ds(wid * ROWS_PER_W, ROWS_PER_W)], gbuf)
        pltpu.sync_copy(sidx.at[pl.ds(wid * ROWS_PER_W, ROWS_PER_W)], sbuf)
        plsc.subcore_barrier()

        def body(j, carry):
            pltpu.sync_copy(src.at[gbuf.at[j]], rows)
            pltpu.sync_copy(rows, acc.at[sbuf.at[j]], add=True)
            return carry

        lax.fori_loop(0, ROWS_PER_W, body, 0)
        plsc.subcore_barrier()
        pltpu.sync_copy(acc.at[pl.ds(sid * rpt, rpt)],
                        out.at[pl.ds(cid * T + sid * rpt, rpt)])

    return sc_pass


def _make_sc_deg(T):
    rpt = T // NS
    mesh = plsc.VectorSubcoreMesh(**_MESH)

    @functools.partial(
        pl.kernel,
        out_type=jax.ShapeDtypeStruct((2 * T, D), jnp.float32),
        mesh=mesh,
        scratch_types=[
            pltpu.VMEM((ROWS_PER_W, CHUNK), jnp.int32),
            pltpu.VMEM((CHUNK, D), jnp.float32),
            pltpu.VMEM_SHARED((T, D), jnp.float32),
        ],
    )
    def sc_deg(sidx, ones, zeros, out, sbuf, onesbuf, acc):
        cid = lax.axis_index("c")
        sid = lax.axis_index("s")
        wid = sid * NC + cid
        pltpu.sync_copy(zeros.at[pl.ds(sid * rpt, rpt)],
                        acc.at[pl.ds(sid * rpt, rpt)])
        pltpu.sync_copy(sidx.at[pl.ds(wid * ROWS_PER_W, ROWS_PER_W)], sbuf)
        pltpu.sync_copy(ones, onesbuf)
        plsc.subcore_barrier()

        def body(j, carry):
            pltpu.sync_copy(onesbuf, acc.at[sbuf.at[j]], add=True)
            return carry

        lax.fori_loop(0, ROWS_PER_W, body, 0)
        plsc.subcore_barrier()
        pltpu.sync_copy(acc.at[pl.ds(sid * rpt, rpt)],
                        out.at[pl.ds(cid * T + sid * rpt, rpt)])

    return sc_deg


def _tc_prep_body(x, w, b, dv2, hs, dvis):
    dv = dv2[0] + dv2[1]
    s = jnp.where(dv > 0, lax.rsqrt(dv), 0.0)
    h = jnp.dot(x[...], w[...], preferred_element_type=jnp.float32) + b[...]
    hs[...] = h * s
    dvis[...] = s


def _tc_prep(X, W1, b1, dv2):
    return pl.pallas_call(
        _tc_prep_body,
        out_shape=(jax.ShapeDtypeStruct((N, D), jnp.float32),
                   jax.ShapeDtypeStruct((N, 1), jnp.float32)),
    )(X, W1, b1, dv2)


def _tc_edge_body(p2, de2, out):
    de = de2[0] + de2[1]
    dei = jnp.where(de > 0, 1.0 / de, 0.0)
    out[...] = (p2[0] + p2[1]) * dei


def _tc_edge(p2, de2):
    return pl.pallas_call(
        _tc_edge_body,
        out_shape=jax.ShapeDtypeStruct((MP, D), jnp.float32),
    )(p2, de2)


def _tc_l2_body(q2, dvis, w, b, out):
    t = jnp.maximum((q2[0] + q2[1]) * dvis[...], 0.0)
    h2 = jnp.dot(t, w[...], preferred_element_type=jnp.float32) + b[...]
    out[...] = h2 * dvis[...]


def _tc_l2(q2, dvis, W2, b2):
    return pl.pallas_call(
        _tc_l2_body,
        out_shape=jax.ShapeDtypeStruct((N, D), jnp.float32),
    )(q2, dvis, W2, b2)


def _tc_final_body(q2, dvis, wvt, bv, out):
    t = jnp.maximum((q2[0] + q2[1]) * dvis[...], 0.0)
    pooled = jnp.sum(t, axis=0, keepdims=True) * (1.0 / N)
    out[...] = jnp.sum(pooled * wvt[...], axis=1, keepdims=True) + bv[...]


def _tc_final(q2, dvis, wvt, bv):
    return pl.pallas_call(
        _tc_final_body,
        out_shape=jax.ShapeDtypeStruct((1, 1), jnp.float32),
    )(q2, dvis, wvt, bv)


_sc_deg_vert = _make_sc_deg(NP)
_sc_deg_edge = _make_sc_deg(MP)
_sc_pass_edge = _make_sc_pass(MP, NP)
_sc_pass_vert = _make_sc_pass(NP, MP)


def kernel(X, vertex_idx, edge_idx, W1, b1, W2, b2, Wv, bv):
    vi = vertex_idx.astype(jnp.int32)
    ei = edge_idx.astype(jnp.int32)
    vip = jnp.concatenate(
        [vi, jnp.full((NNZP - NNZ,), N, jnp.int32)]).reshape(IDXROWS, CHUNK)
    eip = jnp.concatenate(
        [ei, jnp.full((NNZP - NNZ,), M, jnp.int32)]).reshape(IDXROWS, CHUNK)
    ones_deg = jnp.ones((CHUNK, D), jnp.float32)
    zN = jnp.zeros((NP, D), jnp.float32)
    zM = jnp.zeros((MP, D), jnp.float32)

    dv_out = _sc_deg_vert(vip, ones_deg, zN)
    de_out = _sc_deg_edge(eip, ones_deg, zM)
    dv2 = dv_out.reshape(2, NP, D)[:, :N, :1]
    de2 = de_out.reshape(2, MP, D)[:, :, :1]

    hs, dvis = _tc_prep(X, W1, b1.reshape(1, D), dv2)
    hsp = jnp.pad(hs, ((0, NP - N), (0, 0)))
    p = _sc_pass_edge(hsp, vip, eip, zM).reshape(2, MP, D)
    ef = _tc_edge(p, de2)
    q = _sc_pass_vert(ef, eip, vip, zN).reshape(2, NP, D)[:, :N]

    h2 = _tc_l2(q, dvis, W2, b2.reshape(1, D))
    h2p = jnp.pad(h2, ((0, NP - N), (0, 0)))
    p2 = _sc_pass_edge(h2p, vip, eip, zM).reshape(2, MP, D)
    ef2 = _tc_edge(p2, de2)
    q2 = _sc_pass_vert(ef2, eip, vip, zN).reshape(2, NP, D)[:, :N]

    v = _tc_final(q2, dvis, Wv[:, 0].reshape(1, D), bv.reshape(1, 1))
    return v[0]


# speedup vs baseline: 7.0648x; 2.7642x over previous
import functools

import jax
import jax.numpy as jnp
from jax import lax
from jax.experimental import pallas as pl
from jax.experimental.pallas import tpu as pltpu
from jax.experimental.pallas import tpu_sc as plsc

N = 10000
M = 5000
NNZ = 320000
D = 128

NC = 2
NS = 16
NW = NC * NS

NP = 10112
MP = 5120
CHUNK = 128
IDXROWS = 2560
NNZP = IDXROWS * CHUNK
ROWS_PER_W = IDXROWS // NW
DEGW = 16

_MESH = dict(core_axis_name="c", subcore_axis_name="s")


def _make_sc_pass(T, S):
    rpt = T // NS
    mesh = plsc.VectorSubcoreMesh(**_MESH)

    @functools.partial(
        pl.kernel,
        out_type=jax.ShapeDtypeStruct((2 * T, D), jnp.float32),
        mesh=mesh,
        scratch_types=[
            pltpu.VMEM((ROWS_PER_W, CHUNK), jnp.int32),
            pltpu.VMEM((ROWS_PER_W, CHUNK), jnp.int32),
            pltpu.VMEM((CHUNK, D), jnp.float32),
            pltpu.VMEM_SHARED((T, D), jnp.float32),
        ],
    )
    def sc_pass(src, gidx, sidx, zeros, out, gbuf, sbuf, rows, acc):
        cid = lax.axis_index("c")
        sid = lax.axis_index("s")
        wid = sid * NC + cid
        pltpu.sync_copy(zeros.at[pl.ds(sid * rpt, rpt)],
                        acc.at[pl.ds(sid * rpt, rpt)])
        pltpu.sync_copy(gidx.at[pl.ds(wid * ROWS_PER_W, ROWS_PER_W)], gbuf)
        pltpu.sync_copy(sidx.at[pl.ds(wid * ROWS_PER_W, ROWS_PER_W)], sbuf)
        plsc.subcore_barrier()

        def body(j, carry):
            pltpu.sync_copy(src.at[gbuf.at[j]], rows)
            pltpu.sync_copy(rows, acc.at[sbuf.at[j]], add=True)
            return carry

        lax.fori_loop(0, ROWS_PER_W, body, 0)
        plsc.subcore_barrier()
        pltpu.sync_copy(acc.at[pl.ds(sid * rpt, rpt)],
                        out.at[pl.ds(cid * T + sid * rpt, rpt)])

    return sc_pass


def _make_sc_deg(T):
    rpt = T // NS
    mesh = plsc.VectorSubcoreMesh(**_MESH)

    @functools.partial(
        pl.kernel,
        out_type=jax.ShapeDtypeStruct((2 * T, D), jnp.float32),
        mesh=mesh,
        scratch_types=[
            pltpu.VMEM((ROWS_PER_W, CHUNK), jnp.int32),
            pltpu.VMEM((CHUNK, D), jnp.float32),
            pltpu.VMEM_SHARED((T, D), jnp.float32),
        ],
    )
    def sc_deg(sidx, ones, zeros, out, sbuf, onesbuf, acc):
        cid = lax.axis_index("c")
        sid = lax.axis_index("s")
        wid = sid * NC + cid
        pltpu.sync_copy(zeros.at[pl.ds(sid * rpt, rpt)],
                        acc.at[pl.ds(sid * rpt, rpt)])
        pltpu.sync_copy(sidx.at[pl.ds(wid * ROWS_PER_W, ROWS_PER_W)], sbuf)
        pltpu.sync_copy(ones, onesbuf)
        plsc.subcore_barrier()

        def body(j, carry):
            pltpu.sync_copy(onesbuf, acc.at[sbuf.at[j]], add=True)
            return carry

        lax.fori_loop(0, ROWS_PER_W, body, 0)
        plsc.subcore_barrier()
        pltpu.sync_copy(acc.at[pl.ds(sid * rpt, rpt)],
                        out.at[pl.ds(cid * T + sid * rpt, rpt)])

    return sc_deg


def _tc_prep_body(x, w, b, dv2, hs, dvis):
    dv = dv2[0] + dv2[1]
    s = jnp.where(dv > 0, lax.rsqrt(dv), 0.0)
    h = jnp.dot(x[...], w[...], preferred_element_type=jnp.float32) + b[...]
    hs[...] = h * s
    dvis[...] = s


def _tc_prep(X, W1, b1, dv2):
    return pl.pallas_call(
        _tc_prep_body,
        out_shape=(jax.ShapeDtypeStruct((N, D), jnp.float32),
                   jax.ShapeDtypeStruct((N, 1), jnp.float32)),
    )(X, W1, b1, dv2)


def _tc_edge_body(p2, de2, out):
    de = de2[0] + de2[1]
    dei = jnp.where(de > 0, 1.0 / de, 0.0)
    out[...] = (p2[0] + p2[1]) * dei


def _tc_edge(p2, de2):
    return pl.pallas_call(
        _tc_edge_body,
        out_shape=jax.ShapeDtypeStruct((MP, D), jnp.float32),
    )(p2, de2)


def _tc_l2_body(q2, dvis, w, b, out):
    t = jnp.maximum((q2[0] + q2[1]) * dvis[...], 0.0)
    h2 = jnp.dot(t, w[...], preferred_element_type=jnp.float32) + b[...]
    out[...] = h2 * dvis[...]


def _tc_l2(q2, dvis, W2, b2):
    return pl.pallas_call(
        _tc_l2_body,
        out_shape=jax.ShapeDtypeStruct((N, D), jnp.float32),
    )(q2, dvis, W2, b2)


def _tc_final_body(q2, dvis, wvt, bv, out):
    t = jnp.maximum((q2[0] + q2[1]) * dvis[...], 0.0)
    pooled = jnp.sum(t, axis=0, keepdims=True) * (1.0 / N)
    out[...] = jnp.sum(pooled * wvt[...], axis=1, keepdims=True) + bv[...]


def _tc_final(q2, dvis, wvt, bv):
    return pl.pallas_call(
        _tc_final_body,
        out_shape=jax.ShapeDtypeStruct((1, 1), jnp.float32),
    )(q2, dvis, wvt, bv)


_sc_deg_vert = _make_sc_deg(NP)
_sc_deg_edge = _make_sc_deg(MP)
_sc_pass_edge = _make_sc_pass(MP, NP)
_sc_pass_vert = _make_sc_pass(NP, MP)


def kernel(X, vertex_idx, edge_idx, W1, b1, W2, b2, Wv, bv):
    vi = vertex_idx.astype(jnp.int32)
    ei = edge_idx.astype(jnp.int32)
    pad_ar = jnp.arange(NNZP - NNZ, dtype=jnp.int32)
    vip = jnp.concatenate(
        [vi, N + pad_ar % (NP - N)]).reshape(IDXROWS, CHUNK)
    eip = jnp.concatenate(
        [ei, M + pad_ar % (MP - M)]).reshape(IDXROWS, CHUNK)
    ones_deg = jnp.ones((CHUNK, D), jnp.float32)
    zN = jnp.zeros((NP, D), jnp.float32)
    zM = jnp.zeros((MP, D), jnp.float32)

    dv_out = _sc_deg_vert(vip, ones_deg, zN)
    de_out = _sc_deg_edge(eip, ones_deg, zM)
    dv2 = dv_out.reshape(2, NP, D)[:, :N, :1]
    de2 = de_out.reshape(2, MP, D)[:, :, :1]

    hs, dvis = _tc_prep(X, W1, b1.reshape(1, D), dv2)
    hsp = jnp.pad(hs, ((0, NP - N), (0, 0)))
    p = _sc_pass_edge(hsp, vip, eip, zM).reshape(2, MP, D)
    ef = _tc_edge(p, de2)
    q = _sc_pass_vert(ef, eip, vip, zN).reshape(2, NP, D)[:, :N]

    h2 = _tc_l2(q, dvis, W2, b2.reshape(1, D))
    h2p = jnp.pad(h2, ((0, NP - N), (0, 0)))
    p2 = _sc_pass_edge(h2p, vip, eip, zM).reshape(2, MP, D)
    ef2 = _tc_edge(p2, de2)
    q2 = _sc_pass_vert(ef2, eip, vip, zN).reshape(2, NP, D)[:, :N]

    v = _tc_final(q2, dvis, Wv[:, 0].reshape(1, D), bv.reshape(1, 1))
    return v[0]


# speedup vs baseline: 9.9462x; 1.4078x over previous
import functools

import jax
import jax.numpy as jnp
from jax import lax
from jax.experimental import pallas as pl
from jax.experimental.pallas import tpu as pltpu
from jax.experimental.pallas import tpu_sc as plsc

N = 10000
M = 5000
NNZ = 320000
D = 128

NC = 2
NS = 16
NW = NC * NS

NP = 10112
MP = 5120
CHUNK = 128
IDXROWS = 2560
NNZP = IDXROWS * CHUNK
ROWS_PER_W = IDXROWS // NW
DEGW = 16

_MESH = dict(core_axis_name="c", subcore_axis_name="s")


def _make_sc_pass(T, S):
    rpt = T // NS
    mesh = plsc.VectorSubcoreMesh(**_MESH)

    @functools.partial(
        pl.kernel,
        out_type=jax.ShapeDtypeStruct((2 * T, D), jnp.float32),
        mesh=mesh,
        scratch_types=[
            pltpu.VMEM((ROWS_PER_W // 2, CHUNK), jnp.int32),
            pltpu.VMEM((ROWS_PER_W // 2, CHUNK), jnp.int32),
            pltpu.VMEM((CHUNK, D), jnp.float32),
            pltpu.VMEM((CHUNK, D), jnp.float32),
            pltpu.VMEM_SHARED((T, D), jnp.float32),
            pltpu.SemaphoreType.DMA,
            pltpu.SemaphoreType.DMA,
        ],
    )
    def sc_pass(src, gidx, sidx, zeros, out, gbuf, sbuf, rows0, rows1, acc,
                sem0, sem1):
        cid = lax.axis_index("c")
        sid = lax.axis_index("s")
        wid = sid * NC + cid
        half = ROWS_PER_W // 2
        pltpu.sync_copy(zeros.at[pl.ds(sid * rpt, rpt)],
                        acc.at[pl.ds(sid * rpt, rpt)])
        plsc.subcore_barrier()

        for h in range(2):
            base = wid * ROWS_PER_W + h * half
            pltpu.sync_copy(gidx.at[pl.ds(base, half)], gbuf)
            pltpu.sync_copy(sidx.at[pl.ds(base, half)], sbuf)
            pltpu.async_copy(src.at[gbuf.at[0]], rows0, sem0)

            def body(i, carry):
                j = 2 * i
                pltpu.async_copy(src.at[gbuf.at[j + 1]], rows1, sem1)
                pltpu.make_async_copy(src.at[gbuf.at[j]], rows0, sem0).wait()
                pltpu.sync_copy(rows0, acc.at[sbuf.at[j]], add=True)

                @pl.when(j + 2 < half)
                def _():
                    pltpu.async_copy(src.at[gbuf.at[j + 2]], rows0, sem0)

                pltpu.make_async_copy(src.at[gbuf.at[j + 1]], rows1,
                                      sem1).wait()
                pltpu.sync_copy(rows1, acc.at[sbuf.at[j + 1]], add=True)
                return carry

            lax.fori_loop(0, half // 2, body, 0)
        plsc.subcore_barrier()
        pltpu.sync_copy(acc.at[pl.ds(sid * rpt, rpt)],
                        out.at[pl.ds(cid * T + sid * rpt, rpt)])

    return sc_pass


def _make_sc_deg(T):
    rpt = T // NS
    mesh = plsc.VectorSubcoreMesh(**_MESH)

    @functools.partial(
        pl.kernel,
        out_type=jax.ShapeDtypeStruct((2 * T, D), jnp.float32),
        mesh=mesh,
        scratch_types=[
            pltpu.VMEM((ROWS_PER_W, CHUNK), jnp.int32),
            pltpu.VMEM((CHUNK, D), jnp.float32),
            pltpu.VMEM_SHARED((T, D), jnp.float32),
        ],
    )
    def sc_deg(sidx, ones, zeros, out, sbuf, onesbuf, acc):
        cid = lax.axis_index("c")
        sid = lax.axis_index("s")
        wid = sid * NC + cid
        pltpu.sync_copy(zeros.at[pl.ds(sid * rpt, rpt)],
                        acc.at[pl.ds(sid * rpt, rpt)])
        pltpu.sync_copy(sidx.at[pl.ds(wid * ROWS_PER_W, ROWS_PER_W)], sbuf)
        pltpu.sync_copy(ones, onesbuf)
        plsc.subcore_barrier()

        def body(j, carry):
            pltpu.sync_copy(onesbuf, acc.at[sbuf.at[j]], add=True)
            return carry

        lax.fori_loop(0, ROWS_PER_W, body, 0)
        plsc.subcore_barrier()
        pltpu.sync_copy(acc.at[pl.ds(sid * rpt, rpt)],
                        out.at[pl.ds(cid * T + sid * rpt, rpt)])

    return sc_deg


def _tc_prep_body(x, w, b, dv2, hs, dvis):
    dv = dv2[0] + dv2[1]
    s = jnp.where(dv > 0, lax.rsqrt(dv), 0.0)
    h = jnp.dot(x[...], w[...], preferred_element_type=jnp.float32) + b[...]
    hs[...] = h * s
    dvis[...] = s


def _tc_prep(X, W1, b1, dv2):
    return pl.pallas_call(
        _tc_prep_body,
        out_shape=(jax.ShapeDtypeStruct((N, D), jnp.float32),
                   jax.ShapeDtypeStruct((N, 1), jnp.float32)),
    )(X, W1, b1, dv2)


def _tc_edge_body(p2, de2, out):
    de = de2[0] + de2[1]
    dei = jnp.where(de > 0, 1.0 / de, 0.0)
    out[...] = (p2[0] + p2[1]) * dei


def _tc_edge(p2, de2):
    return pl.pallas_call(
        _tc_edge_body,
        out_shape=jax.ShapeDtypeStruct((MP, D), jnp.float32),
    )(p2, de2)


def _tc_l2_body(q2, dvis, w, b, out):
    t = jnp.maximum((q2[0] + q2[1]) * dvis[...], 0.0)
    h2 = jnp.dot(t, w[...], preferred_element_type=jnp.float32) + b[...]
    out[...] = h2 * dvis[...]


def _tc_l2(q2, dvis, W2, b2):
    return pl.pallas_call(
        _tc_l2_body,
        out_shape=jax.ShapeDtypeStruct((N, D), jnp.float32),
    )(q2, dvis, W2, b2)


def _tc_final_body(q2, dvis, wvt, bv, out):
    t = jnp.maximum((q2[0] + q2[1]) * dvis[...], 0.0)
    pooled = jnp.sum(t, axis=0, keepdims=True) * (1.0 / N)
    out[...] = jnp.sum(pooled * wvt[...], axis=1, keepdims=True) + bv[...]


def _tc_final(q2, dvis, wvt, bv):
    return pl.pallas_call(
        _tc_final_body,
        out_shape=jax.ShapeDtypeStruct((1, 1), jnp.float32),
    )(q2, dvis, wvt, bv)


_sc_deg_vert = _make_sc_deg(NP)
_sc_deg_edge = _make_sc_deg(MP)
_sc_pass_edge = _make_sc_pass(MP, NP)
_sc_pass_vert = _make_sc_pass(NP, MP)


def kernel(X, vertex_idx, edge_idx, W1, b1, W2, b2, Wv, bv):
    vi = vertex_idx.astype(jnp.int32)
    ei = edge_idx.astype(jnp.int32)
    pad_ar = jnp.arange(NNZP - NNZ, dtype=jnp.int32)
    vip = jnp.concatenate(
        [vi, N + pad_ar % (NP - N)]).reshape(IDXROWS, CHUNK)
    eip = jnp.concatenate(
        [ei, M + pad_ar % (MP - M)]).reshape(IDXROWS, CHUNK)
    ones_deg = jnp.ones((CHUNK, D), jnp.float32)
    zN = jnp.zeros((NP, D), jnp.float32)
    zM = jnp.zeros((MP, D), jnp.float32)

    dv_out = _sc_deg_vert(vip, ones_deg, zN)
    de_out = _sc_deg_edge(eip, ones_deg, zM)
    dv2 = dv_out.reshape(2, NP, D)[:, :N, :1]
    de2 = de_out.reshape(2, MP, D)[:, :, :1]

    hs, dvis = _tc_prep(X, W1, b1.reshape(1, D), dv2)
    hsp = jnp.pad(hs, ((0, NP - N), (0, 0)))
    p = _sc_pass_edge(hsp, vip, eip, zM).reshape(2, MP, D)
    ef = _tc_edge(p, de2)
    q = _sc_pass_vert(ef, eip, vip, zN).reshape(2, NP, D)[:, :N]

    h2 = _tc_l2(q, dvis, W2, b2.reshape(1, D))
    h2p = jnp.pad(h2, ((0, NP - N), (0, 0)))
    p2 = _sc_pass_edge(h2p, vip, eip, zM).reshape(2, MP, D)
    ef2 = _tc_edge(p2, de2)
    q2 = _sc_pass_vert(ef2, eip, vip, zN).reshape(2, NP, D)[:, :N]

    v = _tc_final(q2, dvis, Wv[:, 0].reshape(1, D), bv.reshape(1, 1))
    return v[0]


# speedup vs baseline: 10.1006x; 1.0155x over previous
import functools

import jax
import jax.numpy as jnp
from jax import lax
from jax.experimental import pallas as pl
from jax.experimental.pallas import tpu as pltpu
from jax.experimental.pallas import tpu_sc as plsc

N = 10000
M = 5000
NNZ = 320000
D = 128

NC = 2
NS = 16
NW = NC * NS

NP = 10112
MP = 5120
CHUNK = 128
IDXROWS = 2560
NNZP = IDXROWS * CHUNK
ROWS_PER_W = IDXROWS // NW
DEGW = 16

_MESH = dict(core_axis_name="c", subcore_axis_name="s")


def _make_sc_pass(T, S):
    rpt = T // NS
    mesh = plsc.VectorSubcoreMesh(**_MESH)

    @functools.partial(
        pl.kernel,
        out_type=jax.ShapeDtypeStruct((2 * T, D), jnp.float32),
        mesh=mesh,
        scratch_types=[
            pltpu.VMEM((ROWS_PER_W // 2, CHUNK), jnp.int32),
            pltpu.VMEM((ROWS_PER_W // 2, CHUNK), jnp.int32),
            pltpu.VMEM((CHUNK, D), jnp.float32),
            pltpu.VMEM((CHUNK, D), jnp.float32),
            pltpu.VMEM_SHARED((T, D), jnp.float32),
            pltpu.SemaphoreType.DMA,
            pltpu.SemaphoreType.DMA,
        ],
    )
    def sc_pass(src, gidx, sidx, zeros, out, gbuf, sbuf, rows0, rows1, acc,
                sem0, sem1):
        cid = lax.axis_index("c")
        sid = lax.axis_index("s")
        wid = sid * NC + cid
        half = ROWS_PER_W // 2
        pltpu.sync_copy(zeros.at[pl.ds(sid * rpt, rpt)],
                        acc.at[pl.ds(sid * rpt, rpt)])
        plsc.subcore_barrier()

        for h in range(2):
            base = wid * ROWS_PER_W + h * half
            pltpu.sync_copy(gidx.at[pl.ds(base, half)], gbuf)
            pltpu.sync_copy(sidx.at[pl.ds(base, half)], sbuf)
            pltpu.async_copy(src.at[gbuf.at[0]], rows0, sem0)

            def body(i, carry):
                j = 2 * i
                pltpu.async_copy(src.at[gbuf.at[j + 1]], rows1, sem1)
                pltpu.make_async_copy(src.at[gbuf.at[j]], rows0, sem0).wait()
                pltpu.sync_copy(rows0, acc.at[sbuf.at[j]], add=True)

                @pl.when(j + 2 < half)
                def _():
                    pltpu.async_copy(src.at[gbuf.at[j + 2]], rows0, sem0)

                pltpu.make_async_copy(src.at[gbuf.at[j + 1]], rows1,
                                      sem1).wait()
                pltpu.sync_copy(rows1, acc.at[sbuf.at[j + 1]], add=True)
                return carry

            lax.fori_loop(0, half // 2, body, 0)
        plsc.subcore_barrier()
        pltpu.sync_copy(acc.at[pl.ds(sid * rpt, rpt)],
                        out.at[pl.ds(cid * T + sid * rpt, rpt)])

    return sc_pass


def _make_sc_deg(T):
    rpt = T // NS
    mesh = plsc.VectorSubcoreMesh(**_MESH)

    @functools.partial(
        pl.kernel,
        out_type=jax.ShapeDtypeStruct((2 * T, D), jnp.float32),
        mesh=mesh,
        scratch_types=[
            pltpu.VMEM((ROWS_PER_W, CHUNK), jnp.int32),
            pltpu.VMEM((CHUNK, D), jnp.float32),
            pltpu.VMEM_SHARED((T, D), jnp.float32),
            pltpu.SemaphoreType.DMA,
        ],
    )
    def sc_deg(sidx, ones, zeros, out, sbuf, onesbuf, acc, sem):
        cid = lax.axis_index("c")
        sid = lax.axis_index("s")
        wid = sid * NC + cid
        pltpu.sync_copy(zeros.at[pl.ds(sid * rpt, rpt)],
                        acc.at[pl.ds(sid * rpt, rpt)])
        pltpu.sync_copy(sidx.at[pl.ds(wid * ROWS_PER_W, ROWS_PER_W)], sbuf)
        pltpu.sync_copy(ones, onesbuf)
        plsc.subcore_barrier()

        K = 8

        def body(i, carry):
            descs = [pltpu.async_copy(onesbuf, acc.at[sbuf.at[i * K + k]],
                                      sem, add=True) for k in range(K)]
            for d in descs:
                d.wait()
            return carry

        lax.fori_loop(0, ROWS_PER_W // K, body, 0)
    return sc_deg


def _tc_prep_body(x, w, b, dv2, hs, dvis):
    dv = dv2[0] + dv2[1]
    s = jnp.where(dv > 0, lax.rsqrt(dv), 0.0)
    h = jnp.dot(x[...], w[...], preferred_element_type=jnp.float32) + b[...]
    hs[...] = h * s
    dvis[...] = s


def _tc_prep(X, W1, b1, dv2):
    return pl.pallas_call(
        _tc_prep_body,
        out_shape=(jax.ShapeDtypeStruct((N, D), jnp.float32),
                   jax.ShapeDtypeStruct((N, 1), jnp.float32)),
    )(X, W1, b1, dv2)


def _tc_edge_body(p2, de2, out):
    de = de2[0] + de2[1]
    dei = jnp.where(de > 0, 1.0 / de, 0.0)
    out[...] = (p2[0] + p2[1]) * dei


def _tc_edge(p2, de2):
    return pl.pallas_call(
        _tc_edge_body,
        out_shape=jax.ShapeDtypeStruct((MP, D), jnp.float32),
    )(p2, de2)


def _tc_l2_body(q2, dvis, w, b, out):
    t = jnp.maximum((q2[0] + q2[1]) * dvis[...], 0.0)
    h2 = jnp.dot(t, w[...], preferred_element_type=jnp.float32) + b[...]
    out[...] = h2 * dvis[...]


def _tc_l2(q2, dvis, W2, b2):
    return pl.pallas_call(
        _tc_l2_body,
        out_shape=jax.ShapeDtypeStruct((N, D), jnp.float32),
    )(q2, dvis, W2, b2)


def _tc_final_body(q2, dvis, wvt, bv, out):
    t = jnp.maximum((q2[0] + q2[1]) * dvis[...], 0.0)
    pooled = jnp.sum(t, axis=0, keepdims=True) * (1.0 / N)
    out[...] = jnp.sum(pooled * wvt[...], axis=1, keepdims=True) + bv[...]


def _tc_final(q2, dvis, wvt, bv):
    return pl.pallas_call(
        _tc_final_body,
        out_shape=jax.ShapeDtypeStruct((1, 1), jnp.float32),
    )(q2, dvis, wvt, bv)


_sc_deg_vert = _make_sc_deg(NP)
_sc_deg_edge = _make_sc_deg(MP)
_sc_pass_edge = _make_sc_pass(MP, NP)
_sc_pass_vert = _make_sc_pass(NP, MP)


def kernel(X, vertex_idx, edge_idx, W1, b1, W2, b2, Wv, bv):
    vi = vertex_idx.astype(jnp.int32)
    ei = edge_idx.astype(jnp.int32)
    pad_ar = jnp.arange(NNZP - NNZ, dtype=jnp.int32)
    vip = jnp.concatenate(
        [vi, N + pad_ar % (NP - N)]).reshape(IDXROWS, CHUNK)
    eip = jnp.concatenate(
        [ei, M + pad_ar % (MP - M)]).reshape(IDXROWS, CHUNK)
    ones_deg = jnp.ones((CHUNK, D), jnp.float32)
    zN = jnp.zeros((NP, D), jnp.float32)
    zM = jnp.zeros((MP, D), jnp.float32)

    dv_out = _sc_deg_vert(vip, ones_deg, zN)
    de_out = _sc_deg_edge(eip, ones_deg, zM)
    dv2 = dv_out.reshape(2, NP, D)[:, :N, :1]
    de2 = de_out.reshape(2, MP, D)[:, :, :1]

    hs, dvis = _tc_prep(X, W1, b1.reshape(1, D), dv2)
    hsp = jnp.pad(hs, ((0, NP - N), (0, 0)))
    p = _sc_pass_edge(hsp, vip, eip, zM).reshape(2, MP, D)
    ef = _tc_edge(p, de2)
    q = _sc_pass_vert(ef, eip, vip, zN).reshape(2, NP, D)[:, :N]

    h2 = _tc_l2(q, dvis, W2, b2.reshape(1, D))
    h2p = jnp.pad(h2, ((0, NP - N), (0, 0)))
    p2 = _sc_pass_edge(h2p, vip, eip, zM).reshape(2, MP, D)
    ef2 = _tc_edge(p2, de2)
    q2 = _sc_pass_vert(ef2, eip, vip, zN).reshape(2, NP, D)[:, :N]

    v = _tc_final(q2, dvis, Wv[:, 0].reshape(1, D), bv.reshape(1, 1))
    return v[0]
